# Initial kernel scaffold; baseline (speedup 1.0000x reference)
import jax
import jax.numpy as jnp
from jax.experimental import pallas as pl


def kernel(u2i, i2u, x_user, x_item, w_user0, w_item0, au_src0, au_dst0, ai_src0, ai_dst0, w_user1, w_item1, au_src1, au_dst1, ai_src1, ai_dst1):
    raise NotImplementedError("write your pallas kernel here")


# speedup vs baseline: 15.2557x; 15.2557x over previous
import functools

import jax
import jax.numpy as jnp
from jax import lax
from jax.experimental import pallas as pl
from jax.experimental.pallas import tpu as pltpu
from jax.experimental.pallas import tpu_sc as plsc

F32 = jnp.float32
I32 = jnp.int32

NU = 5000
NI = 5000
EDG = 160000
D = 128
NPAD = 5120
NC = 2
NS = 16
EW = 10016
EPAD = NS * EW
BLK = EW // 2
CHUNKS = BLK // 16
ACC_W = 272
ROWS_PER_TILE = NPAD // NS
BR = 640
EPS = 1e-16


def _sweep(si_hbm, du_hbm, stab_hbm, dtab_hbm, hsrc_hbm,
           si_v, du_v, stab_v, dtab_v, rows_v, buf_v, acc_sh, sem, s):
  pltpu.sync_copy(stab_hbm, stab_v)
  pltpu.sync_copy(dtab_hbm, dtab_v)

  c256 = jnp.full((16,), 256, I32)
  c257 = jnp.full((16,), 257, I32)
  iota16 = lax.iota(I32, 16)

  def step(j, carry):
    off = j * 16
    vsi = si_v[pl.ds(off, 16)]
    vdu = du_v[pl.ds(off, 16)]
    vsi2 = vsi * 2
    vdu2 = vdu * 2
    ps0 = plsc.load_gather(stab_v, [vsi2])
    ps1 = plsc.load_gather(stab_v, [vsi2 + 1])
    qd0 = plsc.load_gather(dtab_v, [vdu2])
    qd1 = plsc.load_gather(dtab_v, [vdu2 + 1])
    e0 = ps0 + qd0
    e1 = ps1 + qd1
    w0 = jnp.exp(jnp.where(e0 >= 0.0, e0, e0 * 0.01))
    w1 = jnp.exp(jnp.where(e1 >= 0.0, e1, e1 * 0.01))
    pltpu.async_copy(hsrc_hbm.at[vsi], rows_v, sem).wait()
    for e in range(16):
      s0 = w0[e]
      s1 = w1[e]
      for f in range(8):
        r = rows_v[e, pl.ds(f * 16, 16)]
        buf_v[e, pl.ds(f * 16, 16)] = r * s0
        buf_v[e, pl.ds(128 + f * 16, 16)] = r * s1
    plsc.store_scatter(buf_v, [iota16, c256], w0)
    plsc.store_scatter(buf_v, [iota16, c257], w1)
    pltpu.sync_copy(buf_v, acc_sh.at[vdu], add=True)
    return carry

  for b in range(EW // BLK):
    base = s * EW + b * BLK
    pltpu.sync_copy(si_hbm.at[pl.ds(base, BLK)], si_v)
    pltpu.sync_copy(du_hbm.at[pl.ds(base, BLK)], du_v)
    lax.fori_loop(0, CHUNKS, step, 0)


def _sc_layer_body(si0, du0, si1, du1, stab0, dtab0, hsrc0, stab1, dtab1,
                   hsrc1, out, si_v, du_v, stab_v, dtab_v, rows_v, buf_v,
                   acc_sh, sem):
  c = lax.axis_index("c")
  s = lax.axis_index("s")

  z16 = jnp.zeros((16,), F32)
  for e in range(16):
    for jj in range(ACC_W // 16):
      buf_v[e, pl.ds(jj * 16, 16)] = z16
  for k in range(ROWS_PER_TILE // 16):
    pltpu.sync_copy(buf_v, acc_sh.at[pl.ds(s * ROWS_PER_TILE + k * 16, 16)])
  plsc.subcore_barrier()

  @pl.when(c == 0)
  def _():
    _sweep(si0, du0, stab0, dtab0, hsrc0,
           si_v, du_v, stab_v, dtab_v, rows_v, buf_v, acc_sh, sem, s)

  @pl.when(c == 1)
  def _():
    _sweep(si1, du1, stab1, dtab1, hsrc1,
           si_v, du_v, stab_v, dtab_v, rows_v, buf_v, acc_sh, sem, s)

  plsc.subcore_barrier()
  pltpu.sync_copy(acc_sh.at[pl.ds(s * ROWS_PER_TILE, ROWS_PER_TILE)],
                  out.at[c, pl.ds(s * ROWS_PER_TILE, ROWS_PER_TILE)])


@functools.cache
def _make_sc_layer():
  mesh = plsc.VectorSubcoreMesh(core_axis_name="c", subcore_axis_name="s",
                                num_cores=NC, num_subcores=NS)
  return pl.kernel(
      _sc_layer_body,
      out_type=jax.ShapeDtypeStruct((2, NPAD, ACC_W), F32),
      mesh=mesh,
      compiler_params=pltpu.CompilerParams(needs_layout_passes=False, use_tc_tiling_on_sc=False),
      scratch_types=[
          pltpu.VMEM((BLK,), I32),
          pltpu.VMEM((BLK,), I32),
          pltpu.VMEM((NPAD * 2,), F32),
          pltpu.VMEM((NPAD * 2,), F32),
          pltpu.VMEM((16, D), F32),
          pltpu.VMEM((16, ACC_W), F32),
          pltpu.VMEM_SHARED((NPAD, ACC_W), F32),
          pltpu.SemaphoreType.DMA,
      ],
  )


def _sc_layer(*args):
  return _make_sc_layer()(*args)


def _mm_body(x_ref, w_ref, at_ref, h_ref, s_ref):
  h = jnp.dot(x_ref[0], w_ref[0], preferred_element_type=F32)
  h_ref[0] = h
  s_ref[0] = jnp.dot(h, at_ref[0], preferred_element_type=F32)


def _tc_h_scores(xs, ws, ats):
  k = xs.shape[2]
  return pl.pallas_call(
      _mm_body,
      grid=(2, NPAD // BR),
      in_specs=[
          pl.BlockSpec((1, BR, k), lambda t, r: (t, r, 0)),
          pl.BlockSpec((1, k, 128), lambda t, r: (t, 0, 0)),
          pl.BlockSpec((1, 128, 128), lambda t, r: (t, 0, 0)),
      ],
      out_specs=[
          pl.BlockSpec((1, BR, 128), lambda t, r: (t, r, 0)),
          pl.BlockSpec((1, BR, 128), lambda t, r: (t, r, 0)),
      ],
      out_shape=[
          jax.ShapeDtypeStruct((2, NPAD, 128), F32),
          jax.ShapeDtypeStruct((2, NPAD, 128), F32),
      ],
  )(xs, ws, ats)


def _combine_body(acc_ref, bar_ref):
  p = acc_ref[0]
  z0 = p[:, 256:257]
  z1 = p[:, 257:258]
  a0 = p[:, 0:128] / (z0 + EPS)
  a1 = p[:, 128:256] / (z1 + EPS)
  bar_ref[0] = jnp.maximum(jnp.concatenate([a0, a1], axis=1), 0.0)


def _tc_combine(acc):
  return pl.pallas_call(
      _combine_body,
      grid=(2, NPAD // BR),
      in_specs=[pl.BlockSpec((1, BR, ACC_W), lambda t, r: (t, r, 0))],
      out_specs=pl.BlockSpec((1, BR, 256), lambda t, r: (t, r, 0)),
      out_shape=jax.ShapeDtypeStruct((2, NPAD, 256), F32),
  )(acc)


def _final_body(acc_ref, bar_ref):
  p = acc_ref[0]
  z0 = p[:, 256:257]
  z1 = p[:, 257:258]
  a0 = p[:, 0:128] / (z0 + EPS)
  a1 = p[:, 128:256] / (z1 + EPS)
  bar_ref[0] = jnp.maximum((a0 + a1) * 0.5, 0.0)


def _tc_final(acc):
  return pl.pallas_call(
      _final_body,
      grid=(2, NPAD // BR),
      in_specs=[pl.BlockSpec((1, BR, ACC_W), lambda t, r: (t, r, 0))],
      out_specs=pl.BlockSpec((1, BR, 128), lambda t, r: (t, r, 0)),
      out_shape=jax.ShapeDtypeStruct((2, NPAD, 128), F32),
  )(acc)


def _pad_rows(x, n):
  return jnp.concatenate([x, jnp.zeros((n - x.shape[0],) + x.shape[1:], x.dtype)], 0)


def _at_pad(a, b):
  m = jnp.concatenate([a, b], 0).T
  return jnp.concatenate([m, jnp.zeros((m.shape[0], 128 - m.shape[1]), F32)], 1)


def _pad_edges(src, dst):
  npd = EPAD - EDG
  si = jnp.concatenate([src, jnp.zeros((npd,), I32)])
  du = jnp.concatenate([dst, jnp.full((npd,), NU, I32)])
  return si, du


def kernel(u2i, i2u, x_user, x_item, w_user0, w_item0, au_src0, au_dst0,
           ai_src0, ai_dst0, w_user1, w_item1, au_src1, au_dst1, ai_src1,
           ai_dst1):
  xs = jnp.stack([_pad_rows(x_user, NPAD), _pad_rows(x_item, NPAD)])
  w0s = jnp.stack([w_user0, w_item0])
  at0 = jnp.stack([_at_pad(au_dst0, ai_src0), _at_pad(au_src0, ai_dst0)])
  h0, sc0 = _tc_h_scores(xs, w0s, at0)

  si0, du0 = _pad_edges(i2u[0], i2u[1])
  si1, du1 = _pad_edges(u2i[0], u2i[1])

  def run_layer(h, sc):
    return _sc_layer(
        si0, du0, si1, du1,
        sc[1, :, 0:2].reshape(-1), sc[0, :, 0:2].reshape(-1), h[1],
        sc[0, :, 2:4].reshape(-1), sc[1, :, 2:4].reshape(-1), h[0])

  acc0 = run_layer(h0, sc0)
  bar0 = _tc_combine(acc0)

  w1s = jnp.stack([w_user1[256:], w_item1[256:]])
  at1 = jnp.stack([_at_pad(au_dst1, ai_src1), _at_pad(au_src1, ai_dst1)])
  h1, sc1 = _tc_h_scores(bar0, w1s, at1)

  acc1 = run_layer(h1, sc1)
  bar1 = _tc_final(acc1)

  u_bar0 = bar0[0, :NU]
  i_bar0 = bar0[1, :NI]
  u_bar1 = bar1[0, :NU]
  i_bar1 = bar1[1, :NI]
  zu = jnp.zeros((NU, D), F32)
  zi = jnp.zeros((NI, D), F32)
  u = jnp.concatenate([zu, u_bar1], axis=1)
  i = jnp.concatenate([zi, i_bar1], axis=1)
  return (u, i, u_bar0, i_bar0, u_bar1, i_bar1)


# speedup vs baseline: 22.9963x; 1.5074x over previous
import functools

import jax
import jax.numpy as jnp
from jax import lax
from jax.experimental import pallas as pl
from jax.experimental.pallas import tpu as pltpu
from jax.experimental.pallas import tpu_sc as plsc

F32 = jnp.float32
I32 = jnp.int32

NU = 5000
NI = 5000
EDG = 160000
D = 128
NPAD = 5120
NC = 2
NS = 16
EW = 10240
EPAD = NS * EW
BLK = 2560
HALF = BLK // 32
ACC_W = 272
ROWS_PER_TILE = NPAD // NS
BR = 640
EPS = 1e-16


def _sweep(si_hbm, du_hbm, stab_hbm, dtab_hbm, hsrc_hbm,
           si_v, du_v, stab_v, dtab_v, rows_a, rows_b, buf_a, buf_b,
           acc_sh, gsem_a, gsem_b, ssem_a, ssem_b, s):
  pltpu.sync_copy(stab_hbm, stab_v)
  pltpu.sync_copy(dtab_hbm, dtab_v)

  c256 = jnp.full((16,), 256, I32)
  c257 = jnp.full((16,), 257, I32)
  iota16 = lax.iota(I32, 16)

  def weights(vsi, vdu):
    vsi2 = vsi * 2
    vdu2 = vdu * 2
    e0 = plsc.load_gather(stab_v, [vsi2]) + plsc.load_gather(dtab_v, [vdu2])
    e1 = plsc.load_gather(stab_v, [vsi2 + 1]) + plsc.load_gather(dtab_v, [vdu2 + 1])
    w0 = jnp.exp(jnp.where(e0 >= 0.0, e0, e0 * 0.01))
    w1 = jnp.exp(jnp.where(e1 >= 0.0, e1, e1 * 0.01))
    return w0, w1

  def scale(rows_v, buf_v, w0, w1):
    for e in range(16):
      s0 = w0[e]
      s1 = w1[e]
      for f in range(8):
        r = rows_v[e, pl.ds(f * 16, 16)]
        buf_v[e, pl.ds(f * 16, 16)] = r * s0
        buf_v[e, pl.ds(128 + f * 16, 16)] = r * s1
    plsc.store_scatter(buf_v, [iota16, c256], w0)
    plsc.store_scatter(buf_v, [iota16, c257], w1)

  for b in range(EW // BLK):
    base = s * EW + b * BLK
    pltpu.sync_copy(si_hbm.at[pl.ds(base, BLK)], si_v)
    pltpu.sync_copy(du_hbm.at[pl.ds(base, BLK)], du_v)
    pltpu.async_copy(hsrc_hbm.at[si_v[pl.ds(0, 16)]], rows_a, gsem_a)
    pltpu.async_copy(hsrc_hbm.at[si_v[pl.ds(16, 16)]], rows_b, gsem_b)
    first_block = b == 0

    def slot(i, off, rows_v, buf_v, gsem, ssem):
      vsi = si_v[pl.ds(off, 16)]
      vdu = du_v[pl.ds(off, 16)]
      w0, w1 = weights(vsi, vdu)
      pltpu.make_async_copy(hsrc_hbm.at[vsi], rows_v, gsem).wait()

      def wait_buf():
        pltpu.make_async_copy(buf_v, acc_sh.at[iota16], ssem).wait()

      if first_block:
        pl.when(i > 0)(wait_buf)
      else:
        wait_buf()
      scale(rows_v, buf_v, w0, w1)

      @pl.when(i < HALF - 1)
      def _():
        nsi = si_v[pl.ds(off + 32, 16)]
        pltpu.async_copy(hsrc_hbm.at[nsi], rows_v, gsem)

      pltpu.async_copy(buf_v, acc_sh.at[vdu], ssem, add=True)

    def pair(i, carry):
      slot(i, i * 32, rows_a, buf_a, gsem_a, ssem_a)
      slot(i, i * 32 + 16, rows_b, buf_b, gsem_b, ssem_b)
      return carry

    lax.fori_loop(0, HALF, pair, 0)

  pltpu.make_async_copy(buf_a, acc_sh.at[iota16], ssem_a).wait()
  pltpu.make_async_copy(buf_b, acc_sh.at[iota16], ssem_b).wait()


def _sc_layer_body(si0, du0, si1, du1, stab0, dtab0, hsrc0, stab1, dtab1,
                   hsrc1, out, si_v, du_v, stab_v, dtab_v, rows_a, rows_b,
                   buf_a, buf_b, acc_sh, gsem_a, gsem_b, ssem_a, ssem_b):
  c = lax.axis_index("c")
  s = lax.axis_index("s")

  z16 = jnp.zeros((16,), F32)
  for e in range(16):
    for jj in range(ACC_W // 16):
      buf_a[e, pl.ds(jj * 16, 16)] = z16
  for k in range(ROWS_PER_TILE // 16):
    pltpu.sync_copy(buf_a, acc_sh.at[pl.ds(s * ROWS_PER_TILE + k * 16, 16)])
  plsc.subcore_barrier()

  @pl.when(c == 0)
  def _():
    _sweep(si0, du0, stab0, dtab0, hsrc0,
           si_v, du_v, stab_v, dtab_v, rows_a, rows_b, buf_a, buf_b,
           acc_sh, gsem_a, gsem_b, ssem_a, ssem_b, s)

  @pl.when(c == 1)
  def _():
    _sweep(si1, du1, stab1, dtab1, hsrc1,
           si_v, du_v, stab_v, dtab_v, rows_a, rows_b, buf_a, buf_b,
           acc_sh, gsem_a, gsem_b, ssem_a, ssem_b, s)

  plsc.subcore_barrier()
  pltpu.sync_copy(acc_sh.at[pl.ds(s * ROWS_PER_TILE, ROWS_PER_TILE)],
                  out.at[c, pl.ds(s * ROWS_PER_TILE, ROWS_PER_TILE)])


@functools.cache
def _make_sc_layer():
  mesh = plsc.VectorSubcoreMesh(core_axis_name="c", subcore_axis_name="s",
                                num_cores=NC, num_subcores=NS)
  return pl.kernel(
      _sc_layer_body,
      out_type=jax.ShapeDtypeStruct((2, NPAD, ACC_W), F32),
      mesh=mesh,
      compiler_params=pltpu.CompilerParams(needs_layout_passes=False, use_tc_tiling_on_sc=False),
      scratch_types=[
          pltpu.VMEM((BLK,), I32),
          pltpu.VMEM((BLK,), I32),
          pltpu.VMEM((NPAD * 2,), F32),
          pltpu.VMEM((NPAD * 2,), F32),
          pltpu.VMEM((16, D), F32),
          pltpu.VMEM((16, D), F32),
          pltpu.VMEM((16, ACC_W), F32),
          pltpu.VMEM((16, ACC_W), F32),
          pltpu.VMEM_SHARED((NPAD, ACC_W), F32),
          pltpu.SemaphoreType.DMA,
          pltpu.SemaphoreType.DMA,
          pltpu.SemaphoreType.DMA,
          pltpu.SemaphoreType.DMA,
      ],
  )


def _sc_layer(*args):
  return _make_sc_layer()(*args)


def _mm_body(x_ref, w_ref, at_ref, h_ref, s_ref):
  h = jnp.dot(x_ref[0], w_ref[0], preferred_element_type=F32)
  h_ref[0] = h
  s_ref[0] = jnp.dot(h, at_ref[0], preferred_element_type=F32)


def _tc_h_scores(xs, ws, ats):
  k = xs.shape[2]
  return pl.pallas_call(
      _mm_body,
      grid=(2, NPAD // BR),
      in_specs=[
          pl.BlockSpec((1, BR, k), lambda t, r: (t, r, 0)),
          pl.BlockSpec((1, k, 128), lambda t, r: (t, 0, 0)),
          pl.BlockSpec((1, 128, 128), lambda t, r: (t, 0, 0)),
      ],
      out_specs=[
          pl.BlockSpec((1, BR, 128), lambda t, r: (t, r, 0)),
          pl.BlockSpec((1, BR, 128), lambda t, r: (t, r, 0)),
      ],
      out_shape=[
          jax.ShapeDtypeStruct((2, NPAD, 128), F32),
          jax.ShapeDtypeStruct((2, NPAD, 128), F32),
      ],
  )(xs, ws, ats)


def _combine_body(acc_ref, bar_ref):
  p = acc_ref[0]
  z0 = p[:, 256:257]
  z1 = p[:, 257:258]
  a0 = p[:, 0:128] / (z0 + EPS)
  a1 = p[:, 128:256] / (z1 + EPS)
  bar_ref[0] = jnp.maximum(jnp.concatenate([a0, a1], axis=1), 0.0)


def _tc_combine(acc):
  return pl.pallas_call(
      _combine_body,
      grid=(2, NPAD // BR),
      in_specs=[pl.BlockSpec((1, BR, ACC_W), lambda t, r: (t, r, 0))],
      out_specs=pl.BlockSpec((1, BR, 256), lambda t, r: (t, r, 0)),
      out_shape=jax.ShapeDtypeStruct((2, NPAD, 256), F32),
  )(acc)


def _final_body(acc_ref, bar_ref):
  p = acc_ref[0]
  z0 = p[:, 256:257]
  z1 = p[:, 257:258]
  a0 = p[:, 0:128] / (z0 + EPS)
  a1 = p[:, 128:256] / (z1 + EPS)
  bar_ref[0] = jnp.maximum((a0 + a1) * 0.5, 0.0)


def _tc_final(acc):
  return pl.pallas_call(
      _final_body,
      grid=(2, NPAD // BR),
      in_specs=[pl.BlockSpec((1, BR, ACC_W), lambda t, r: (t, r, 0))],
      out_specs=pl.BlockSpec((1, BR, 128), lambda t, r: (t, r, 0)),
      out_shape=jax.ShapeDtypeStruct((2, NPAD, 128), F32),
  )(acc)


def _pad_rows(x, n):
  return jnp.concatenate([x, jnp.zeros((n - x.shape[0],) + x.shape[1:], x.dtype)], 0)


def _at_pad(a, b):
  m = jnp.concatenate([a, b], 0).T
  return jnp.concatenate([m, jnp.zeros((m.shape[0], 128 - m.shape[1]), F32)], 1)


def _pad_edges(src, dst):
  npd = EPAD - EDG
  si = jnp.concatenate([src, jnp.zeros((npd,), I32)])
  du = jnp.concatenate([dst, jnp.full((npd,), NU, I32)])
  return si, du


def kernel(u2i, i2u, x_user, x_item, w_user0, w_item0, au_src0, au_dst0,
           ai_src0, ai_dst0, w_user1, w_item1, au_src1, au_dst1, ai_src1,
           ai_dst1):
  xs = jnp.stack([_pad_rows(x_user, NPAD), _pad_rows(x_item, NPAD)])
  w0s = jnp.stack([w_user0, w_item0])
  at0 = jnp.stack([_at_pad(au_dst0, ai_src0), _at_pad(au_src0, ai_dst0)])
  h0, sc0 = _tc_h_scores(xs, w0s, at0)

  si0, du0 = _pad_edges(i2u[0], i2u[1])
  si1, du1 = _pad_edges(u2i[0], u2i[1])

  def run_layer(h, sc):
    return _sc_layer(
        si0, du0, si1, du1,
        sc[1, :, 0:2].reshape(-1), sc[0, :, 0:2].reshape(-1), h[1],
        sc[0, :, 2:4].reshape(-1), sc[1, :, 2:4].reshape(-1), h[0])

  acc0 = run_layer(h0, sc0)
  bar0 = _tc_combine(acc0)

  w1s = jnp.stack([w_user1[256:], w_item1[256:]])
  at1 = jnp.stack([_at_pad(au_dst1, ai_src1), _at_pad(au_src1, ai_dst1)])
  h1, sc1 = _tc_h_scores(bar0, w1s, at1)

  acc1 = run_layer(h1, sc1)
  bar1 = _tc_final(acc1)

  u_bar0 = bar0[0, :NU]
  i_bar0 = bar0[1, :NI]
  u_bar1 = bar1[0, :NU]
  i_bar1 = bar1[1, :NI]
  zu = jnp.zeros((NU, D), F32)
  zi = jnp.zeros((NI, D), F32)
  u = jnp.concatenate([zu, u_bar1], axis=1)
  i = jnp.concatenate([zi, i_bar1], axis=1)
  return (u, i, u_bar0, i_bar0, u_bar1, i_bar1)


# speedup vs baseline: 25.7234x; 1.1186x over previous
import functools

import jax
import jax.numpy as jnp
from jax import lax
from jax.experimental import pallas as pl
from jax.experimental.pallas import tpu as pltpu
from jax.experimental.pallas import tpu_sc as plsc

F32 = jnp.float32
I32 = jnp.int32

NU = 5000
NI = 5000
EDG = 160000
D = 128
NPAD = 5120
NC = 2
NS = 16
EW = 10240
EPAD = NS * EW
BLK = 2560
QUAD = BLK // 64
ACC_W = 272
ROWS_PER_TILE = NPAD // NS
BR = 640
EPS = 1e-16


def _sweep(si_hbm, du_hbm, stab_hbm, dtab_hbm, hsrc_hbm,
           si_v, du_v, stab_v, dtab_v, rows_a, rows_b, rows_c, rows_d,
           buf_a, buf_b, acc_sh, gsem_a, gsem_b, gsem_c, gsem_d,
           ssem_a, ssem_b, s):
  pltpu.sync_copy(stab_hbm, stab_v)
  pltpu.sync_copy(dtab_hbm, dtab_v)

  c256 = jnp.full((16,), 256, I32)
  c257 = jnp.full((16,), 257, I32)
  iota16 = lax.iota(I32, 16)

  def weights(vsi, vdu):
    vsi2 = vsi * 2
    vdu2 = vdu * 2
    e0 = plsc.load_gather(stab_v, [vsi2]) + plsc.load_gather(dtab_v, [vdu2])
    e1 = plsc.load_gather(stab_v, [vsi2 + 1]) + plsc.load_gather(dtab_v, [vdu2 + 1])
    w0 = jnp.exp(jnp.where(e0 >= 0.0, e0, e0 * 0.01))
    w1 = jnp.exp(jnp.where(e1 >= 0.0, e1, e1 * 0.01))
    return w0, w1

  def scale(rows_v, buf_v, w0, w1):
    s0 = [w0[e] for e in range(16)]
    s1 = [w1[e] for e in range(16)]

    def fbody(f, carry):
      off = f * 16
      for e in range(16):
        r = rows_v[e, pl.ds(off, 16)]
        buf_v[e, pl.ds(off, 16)] = r * s0[e]
        buf_v[e, pl.ds(128 + off, 16)] = r * s1[e]
      return carry

    lax.fori_loop(0, 8, fbody, 0)
    plsc.store_scatter(buf_v, [iota16, c256], w0)
    plsc.store_scatter(buf_v, [iota16, c257], w1)

  rows_ring = [rows_a, rows_b, rows_c, rows_d]
  gsems = [gsem_a, gsem_b, gsem_c, gsem_d]
  bufs = [buf_a, buf_b, buf_a, buf_b]
  ssems = [ssem_a, ssem_b, ssem_a, ssem_b]

  for b in range(EW // BLK):
    base = s * EW + b * BLK
    pltpu.sync_copy(si_hbm.at[pl.ds(base, BLK)], si_v)
    pltpu.sync_copy(du_hbm.at[pl.ds(base, BLK)], du_v)
    for q in range(4):
      pltpu.async_copy(hsrc_hbm.at[si_v[pl.ds(q * 16, 16)]], rows_ring[q],
                       gsems[q])
    first_block = b == 0

    def slot(i, q, rows_v, buf_v, gsem, ssem):
      off = i * 64 + q * 16
      vsi = si_v[pl.ds(off, 16)]
      vdu = du_v[pl.ds(off, 16)]
      w0, w1 = weights(vsi, vdu)
      pltpu.make_async_copy(hsrc_hbm.at[vsi], rows_v, gsem).wait()

      def wait_buf():
        pltpu.make_async_copy(buf_v, acc_sh.at[iota16], ssem).wait()

      if first_block and q < 2:
        pl.when(i > 0)(wait_buf)
      else:
        wait_buf()
      scale(rows_v, buf_v, w0, w1)

      @pl.when(i < QUAD - 1)
      def _():
        nsi = si_v[pl.ds(off + 64, 16)]
        pltpu.async_copy(hsrc_hbm.at[nsi], rows_v, gsem)

      pltpu.async_copy(buf_v, acc_sh.at[vdu], ssem, add=True)

    def quad(i, carry):
      for q in range(4):
        slot(i, q, rows_ring[q], bufs[q], gsems[q], ssems[q])
      return carry

    lax.fori_loop(0, QUAD, quad, 0)

  pltpu.make_async_copy(buf_a, acc_sh.at[iota16], ssem_a).wait()
  pltpu.make_async_copy(buf_b, acc_sh.at[iota16], ssem_b).wait()


def _sc_layer_body(si0, du0, si1, du1, stab0, dtab0, hsrc0, stab1, dtab1,
                   hsrc1, out, si_v, du_v, stab_v, dtab_v, rows_a, rows_b,
                   rows_c, rows_d, buf_a, buf_b, acc_sh, gsem_a, gsem_b,
                   gsem_c, gsem_d, ssem_a, ssem_b):
  c = lax.axis_index("c")
  s = lax.axis_index("s")

  z16 = jnp.zeros((16,), F32)
  for e in range(16):
    for jj in range(ACC_W // 16):
      buf_a[e, pl.ds(jj * 16, 16)] = z16
  for k in range(ROWS_PER_TILE // 16):
    pltpu.sync_copy(buf_a, acc_sh.at[pl.ds(s * ROWS_PER_TILE + k * 16, 16)])
  plsc.subcore_barrier()

  @pl.when(c == 0)
  def _():
    _sweep(si0, du0, stab0, dtab0, hsrc0,
           si_v, du_v, stab_v, dtab_v, rows_a, rows_b, rows_c, rows_d,
           buf_a, buf_b, acc_sh, gsem_a, gsem_b, gsem_c, gsem_d,
           ssem_a, ssem_b, s)

  @pl.when(c == 1)
  def _():
    _sweep(si1, du1, stab1, dtab1, hsrc1,
           si_v, du_v, stab_v, dtab_v, rows_a, rows_b, rows_c, rows_d,
           buf_a, buf_b, acc_sh, gsem_a, gsem_b, gsem_c, gsem_d,
           ssem_a, ssem_b, s)

  plsc.subcore_barrier()
  pltpu.sync_copy(acc_sh.at[pl.ds(s * ROWS_PER_TILE, ROWS_PER_TILE)],
                  out.at[c, pl.ds(s * ROWS_PER_TILE, ROWS_PER_TILE)])


@functools.cache
def _make_sc_layer():
  mesh = plsc.VectorSubcoreMesh(core_axis_name="c", subcore_axis_name="s",
                                num_cores=NC, num_subcores=NS)
  return pl.kernel(
      _sc_layer_body,
      out_type=jax.ShapeDtypeStruct((2, NPAD, ACC_W), F32),
      mesh=mesh,
      compiler_params=pltpu.CompilerParams(needs_layout_passes=False, use_tc_tiling_on_sc=False),
      scratch_types=[
          pltpu.VMEM((BLK,), I32),
          pltpu.VMEM((BLK,), I32),
          pltpu.VMEM((NPAD * 2,), F32),
          pltpu.VMEM((NPAD * 2,), F32),
          pltpu.VMEM((16, D), F32),
          pltpu.VMEM((16, D), F32),
          pltpu.VMEM((16, D), F32),
          pltpu.VMEM((16, D), F32),
          pltpu.VMEM((16, ACC_W), F32),
          pltpu.VMEM((16, ACC_W), F32),
          pltpu.VMEM_SHARED((NPAD, ACC_W), F32),
          pltpu.SemaphoreType.DMA,
          pltpu.SemaphoreType.DMA,
          pltpu.SemaphoreType.DMA,
          pltpu.SemaphoreType.DMA,
          pltpu.SemaphoreType.DMA,
          pltpu.SemaphoreType.DMA,
      ],
  )


def _sc_layer(*args):
  return _make_sc_layer()(*args)


def _mm_body(x_ref, w_ref, at_ref, h_ref, s_ref):
  h = jnp.dot(x_ref[0], w_ref[0], preferred_element_type=F32)
  h_ref[0] = h
  s_ref[0] = jnp.dot(h, at_ref[0], preferred_element_type=F32)


def _tc_h_scores(xs, ws, ats):
  k = xs.shape[2]
  return pl.pallas_call(
      _mm_body,
      grid=(2, NPAD // BR),
      in_specs=[
          pl.BlockSpec((1, BR, k), lambda t, r: (t, r, 0)),
          pl.BlockSpec((1, k, 128), lambda t, r: (t, 0, 0)),
          pl.BlockSpec((1, 128, 128), lambda t, r: (t, 0, 0)),
      ],
      out_specs=[
          pl.BlockSpec((1, BR, 128), lambda t, r: (t, r, 0)),
          pl.BlockSpec((1, BR, 128), lambda t, r: (t, r, 0)),
      ],
      out_shape=[
          jax.ShapeDtypeStruct((2, NPAD, 128), F32),
          jax.ShapeDtypeStruct((2, NPAD, 128), F32),
      ],
  )(xs, ws, ats)


def _combine_body(acc_ref, bar_ref):
  p = acc_ref[0]
  z0 = p[:, 256:257]
  z1 = p[:, 257:258]
  a0 = p[:, 0:128] / (z0 + EPS)
  a1 = p[:, 128:256] / (z1 + EPS)
  bar_ref[0] = jnp.maximum(jnp.concatenate([a0, a1], axis=1), 0.0)


def _tc_combine(acc):
  return pl.pallas_call(
      _combine_body,
      grid=(2, NPAD // BR),
      in_specs=[pl.BlockSpec((1, BR, ACC_W), lambda t, r: (t, r, 0))],
      out_specs=pl.BlockSpec((1, BR, 256), lambda t, r: (t, r, 0)),
      out_shape=jax.ShapeDtypeStruct((2, NPAD, 256), F32),
  )(acc)


def _final_body(acc_ref, bar_ref):
  p = acc_ref[0]
  z0 = p[:, 256:257]
  z1 = p[:, 257:258]
  a0 = p[:, 0:128] / (z0 + EPS)
  a1 = p[:, 128:256] / (z1 + EPS)
  bar_ref[0] = jnp.maximum((a0 + a1) * 0.5, 0.0)


def _tc_final(acc):
  return pl.pallas_call(
      _final_body,
      grid=(2, NPAD // BR),
      in_specs=[pl.BlockSpec((1, BR, ACC_W), lambda t, r: (t, r, 0))],
      out_specs=pl.BlockSpec((1, BR, 128), lambda t, r: (t, r, 0)),
      out_shape=jax.ShapeDtypeStruct((2, NPAD, 128), F32),
  )(acc)


def _pad_rows(x, n):
  return jnp.concatenate([x, jnp.zeros((n - x.shape[0],) + x.shape[1:], x.dtype)], 0)


def _at_pad(a, b):
  m = jnp.concatenate([a, b], 0).T
  return jnp.concatenate([m, jnp.zeros((m.shape[0], 128 - m.shape[1]), F32)], 1)


def _pad_edges(src, dst):
  npd = EPAD - EDG
  si = jnp.concatenate([src, jnp.zeros((npd,), I32)])
  du = jnp.concatenate([dst, jnp.full((npd,), NU, I32)])
  return si, du


def kernel(u2i, i2u, x_user, x_item, w_user0, w_item0, au_src0, au_dst0,
           ai_src0, ai_dst0, w_user1, w_item1, au_src1, au_dst1, ai_src1,
           ai_dst1):
  xs = jnp.stack([_pad_rows(x_user, NPAD), _pad_rows(x_item, NPAD)])
  w0s = jnp.stack([w_user0, w_item0])
  at0 = jnp.stack([_at_pad(au_dst0, ai_src0), _at_pad(au_src0, ai_dst0)])
  h0, sc0 = _tc_h_scores(xs, w0s, at0)

  si0, du0 = _pad_edges(i2u[0], i2u[1])
  si1, du1 = _pad_edges(u2i[0], u2i[1])

  def run_layer(h, sc):
    return _sc_layer(
        si0, du0, si1, du1,
        sc[1, :, 0:2].reshape(-1), sc[0, :, 0:2].reshape(-1), h[1],
        sc[0, :, 2:4].reshape(-1), sc[1, :, 2:4].reshape(-1), h[0])

  acc0 = run_layer(h0, sc0)
  bar0 = _tc_combine(acc0)

  w1s = jnp.stack([w_user1[256:], w_item1[256:]])
  at1 = jnp.stack([_at_pad(au_dst1, ai_src1), _at_pad(au_src1, ai_dst1)])
  h1, sc1 = _tc_h_scores(bar0, w1s, at1)

  acc1 = run_layer(h1, sc1)
  bar1 = _tc_final(acc1)

  u_bar0 = bar0[0, :NU]
  i_bar0 = bar0[1, :NI]
  u_bar1 = bar1[0, :NU]
  i_bar1 = bar1[1, :NI]
  zu = jnp.zeros((NU, D), F32)
  zi = jnp.zeros((NI, D), F32)
  u = jnp.concatenate([zu, u_bar1], axis=1)
  i = jnp.concatenate([zi, i_bar1], axis=1)
  return (u, i, u_bar0, i_bar0, u_bar1, i_bar1)


# speedup vs baseline: 25.8245x; 1.0039x over previous
import functools

import jax
import jax.numpy as jnp
from jax import lax
from jax.experimental import pallas as pl
from jax.experimental.pallas import tpu as pltpu
from jax.experimental.pallas import tpu_sc as plsc

F32 = jnp.float32
I32 = jnp.int32

NU = 5000
NI = 5000
EDG = 160000
D = 128
NPAD = 5120
NC = 2
NS = 16
EW = 10240
EPAD = NS * EW
BLK = 2560
QUAD = BLK // 64
ACC_W = 272
ROWS_PER_TILE = NPAD // NS
BR = 640
EPS = 1e-16


def _sweep(si_hbm, du_hbm, stab_hbm, dtab_hbm, hsrc_hbm,
           si_v, du_v, stab_v, dtab_v, rows_a, rows_b, rows_c, rows_d,
           buf_a, buf_b, acc_sh, gsem_a, gsem_b, gsem_c, gsem_d,
           ssem_a, ssem_b, s):
  pltpu.sync_copy(stab_hbm, stab_v)
  pltpu.sync_copy(dtab_hbm, dtab_v)

  c256 = jnp.full((16,), 256, I32)
  c257 = jnp.full((16,), 257, I32)
  iota16 = lax.iota(I32, 16)

  def weights(vsi, vdu):
    vsi2 = vsi * 2
    vdu2 = vdu * 2
    e0 = plsc.load_gather(stab_v, [vsi2]) + plsc.load_gather(dtab_v, [vdu2])
    e1 = plsc.load_gather(stab_v, [vsi2 + 1]) + plsc.load_gather(dtab_v, [vdu2 + 1])
    w0 = jnp.exp(jnp.where(e0 >= 0.0, e0, e0 * 0.01))
    w1 = jnp.exp(jnp.where(e1 >= 0.0, e1, e1 * 0.01))
    return w0, w1

  def scale(rows_v, buf_v, w0, w1):
    s0 = [w0[e] for e in range(16)]
    s1 = [w1[e] for e in range(16)]

    def fbody(f, carry):
      off = f * 16
      for e in range(16):
        r = rows_v[e, pl.ds(off, 16)]
        buf_v[e, pl.ds(off, 16)] = r * s0[e]
        buf_v[e, pl.ds(128 + off, 16)] = r * s1[e]
      return carry

    lax.fori_loop(0, 8, fbody, 0)
    plsc.store_scatter(buf_v, [iota16, c256], w0)
    plsc.store_scatter(buf_v, [iota16, c257], w1)

  rows_ring = [rows_a, rows_b, rows_c, rows_d]
  gsems = [gsem_a, gsem_b, gsem_c, gsem_d]
  bufs = [buf_a, buf_b, buf_a, buf_b]
  ssems = [ssem_a, ssem_b, ssem_a, ssem_b]

  for b in range(EW // BLK):
    base = s * EW + b * BLK
    pltpu.sync_copy(si_hbm.at[pl.ds(base, BLK)], si_v)
    pltpu.sync_copy(du_hbm.at[pl.ds(base, BLK)], du_v)
    for q in range(4):
      pltpu.async_copy(hsrc_hbm.at[si_v[pl.ds(q * 16, 16)]], rows_ring[q],
                       gsems[q])
    first_block = b == 0

    def slot(i, q, rows_v, buf_v, gsem, ssem):
      off = i * 64 + q * 16
      vsi = si_v[pl.ds(off, 16)]
      vdu = du_v[pl.ds(off, 16)]
      w0, w1 = weights(vsi, vdu)
      pltpu.make_async_copy(hsrc_hbm.at[vsi], rows_v, gsem).wait()

      def wait_buf():
        pltpu.make_async_copy(buf_v, acc_sh.at[iota16], ssem).wait()

      if first_block and q < 2:
        pl.when(i > 0)(wait_buf)
      else:
        wait_buf()
      scale(rows_v, buf_v, w0, w1)

      @pl.when(i < QUAD - 1)
      def _():
        nsi = si_v[pl.ds(off + 64, 16)]
        pltpu.async_copy(hsrc_hbm.at[nsi], rows_v, gsem)

      pltpu.async_copy(buf_v, acc_sh.at[vdu], ssem, add=True)

    def quad(i, carry):
      for q in range(4):
        slot(i, q, rows_ring[q], bufs[q], gsems[q], ssems[q])
      return carry

    lax.fori_loop(0, QUAD, quad, 0)

  pltpu.make_async_copy(buf_a, acc_sh.at[iota16], ssem_a).wait()
  pltpu.make_async_copy(buf_b, acc_sh.at[iota16], ssem_b).wait()


def _sc_layer_body(si0, du0, si1, du1, stab0, dtab0, hsrc0, stab1, dtab1,
                   hsrc1, out, si_v, du_v, stab_v, dtab_v, rows_a, rows_b,
                   rows_c, rows_d, buf_a, buf_b, acc_sh, gsem_a, gsem_b,
                   gsem_c, gsem_d, ssem_a, ssem_b):
  c = lax.axis_index("c")
  s = lax.axis_index("s")

  z16 = jnp.zeros((16,), F32)
  for e in range(16):
    for jj in range(ACC_W // 16):
      buf_a[e, pl.ds(jj * 16, 16)] = z16
  for k in range(ROWS_PER_TILE // 16):
    pltpu.sync_copy(buf_a, acc_sh.at[pl.ds(s * ROWS_PER_TILE + k * 16, 16)])
  plsc.subcore_barrier()

  @pl.when(c == 0)
  def _():
    _sweep(si0, du0, stab0, dtab0, hsrc0,
           si_v, du_v, stab_v, dtab_v, rows_a, rows_b, rows_c, rows_d,
           buf_a, buf_b, acc_sh, gsem_a, gsem_b, gsem_c, gsem_d,
           ssem_a, ssem_b, s)

  @pl.when(c == 1)
  def _():
    _sweep(si1, du1, stab1, dtab1, hsrc1,
           si_v, du_v, stab_v, dtab_v, rows_a, rows_b, rows_c, rows_d,
           buf_a, buf_b, acc_sh, gsem_a, gsem_b, gsem_c, gsem_d,
           ssem_a, ssem_b, s)

  plsc.subcore_barrier()
  pltpu.sync_copy(acc_sh.at[pl.ds(s * ROWS_PER_TILE, ROWS_PER_TILE)],
                  out.at[c, pl.ds(s * ROWS_PER_TILE, ROWS_PER_TILE)])


@functools.cache
def _make_sc_layer():
  mesh = plsc.VectorSubcoreMesh(core_axis_name="c", subcore_axis_name="s",
                                num_cores=NC, num_subcores=NS)
  return pl.kernel(
      _sc_layer_body,
      out_type=jax.ShapeDtypeStruct((2, NPAD, ACC_W), F32),
      mesh=mesh,
      compiler_params=pltpu.CompilerParams(needs_layout_passes=False, use_tc_tiling_on_sc=False),
      scratch_types=[
          pltpu.VMEM((BLK,), I32),
          pltpu.VMEM((BLK,), I32),
          pltpu.VMEM((NPAD * 2,), F32),
          pltpu.VMEM((NPAD * 2,), F32),
          pltpu.VMEM((16, D), F32),
          pltpu.VMEM((16, D), F32),
          pltpu.VMEM((16, D), F32),
          pltpu.VMEM((16, D), F32),
          pltpu.VMEM((16, ACC_W), F32),
          pltpu.VMEM((16, ACC_W), F32),
          pltpu.VMEM_SHARED((NPAD, ACC_W), F32),
          pltpu.SemaphoreType.DMA,
          pltpu.SemaphoreType.DMA,
          pltpu.SemaphoreType.DMA,
          pltpu.SemaphoreType.DMA,
          pltpu.SemaphoreType.DMA,
          pltpu.SemaphoreType.DMA,
      ],
  )


def _sc_layer(*args):
  return _make_sc_layer()(*args)


def _mm_body(x_ref, w_ref, at_ref, h_ref, s_ref):
  h = jnp.dot(x_ref[0], w_ref[0], preferred_element_type=F32)
  h_ref[0] = h
  s_ref[0] = jnp.dot(h, at_ref[0], preferred_element_type=F32)


def _tc_h_scores(xs, ws, ats):
  k = xs.shape[2]
  return pl.pallas_call(
      _mm_body,
      grid=(2, NPAD // BR),
      in_specs=[
          pl.BlockSpec((1, BR, k), lambda t, r: (t, r, 0)),
          pl.BlockSpec((1, k, 128), lambda t, r: (t, 0, 0)),
          pl.BlockSpec((1, 128, 128), lambda t, r: (t, 0, 0)),
      ],
      out_specs=[
          pl.BlockSpec((1, BR, 128), lambda t, r: (t, r, 0)),
          pl.BlockSpec((1, BR, 128), lambda t, r: (t, r, 0)),
      ],
      out_shape=[
          jax.ShapeDtypeStruct((2, NPAD, 128), F32),
          jax.ShapeDtypeStruct((2, NPAD, 128), F32),
      ],
  )(xs, ws, ats)


def _combine_mm_body(acc_ref, w_ref, at_ref, bar_ref, h_ref, s_ref):
  p = acc_ref[0]
  z0 = p[:, 256:257]
  z1 = p[:, 257:258]
  a0 = p[:, 0:128] / (z0 + EPS)
  a1 = p[:, 128:256] / (z1 + EPS)
  bar = jnp.maximum(jnp.concatenate([a0, a1], axis=1), 0.0)
  bar_ref[0] = bar
  h = jnp.dot(bar, w_ref[0], preferred_element_type=F32)
  h_ref[0] = h
  s_ref[0] = jnp.dot(h, at_ref[0], preferred_element_type=F32)


def _tc_combine_mm(acc, ws, ats):
  return pl.pallas_call(
      _combine_mm_body,
      grid=(2, NPAD // BR),
      in_specs=[
          pl.BlockSpec((1, BR, ACC_W), lambda t, r: (t, r, 0)),
          pl.BlockSpec((1, 256, 128), lambda t, r: (t, 0, 0)),
          pl.BlockSpec((1, 128, 128), lambda t, r: (t, 0, 0)),
      ],
      out_specs=[
          pl.BlockSpec((1, BR, 256), lambda t, r: (t, r, 0)),
          pl.BlockSpec((1, BR, 128), lambda t, r: (t, r, 0)),
          pl.BlockSpec((1, BR, 128), lambda t, r: (t, r, 0)),
      ],
      out_shape=[
          jax.ShapeDtypeStruct((2, NPAD, 256), F32),
          jax.ShapeDtypeStruct((2, NPAD, 128), F32),
          jax.ShapeDtypeStruct((2, NPAD, 128), F32),
      ],
  )(acc, ws, ats)


def _final_body(acc_ref, bar_ref):
  p = acc_ref[0]
  z0 = p[:, 256:257]
  z1 = p[:, 257:258]
  a0 = p[:, 0:128] / (z0 + EPS)
  a1 = p[:, 128:256] / (z1 + EPS)
  bar_ref[0] = jnp.maximum((a0 + a1) * 0.5, 0.0)


def _tc_final(acc):
  return pl.pallas_call(
      _final_body,
      grid=(2, NPAD // BR),
      in_specs=[pl.BlockSpec((1, BR, ACC_W), lambda t, r: (t, r, 0))],
      out_specs=pl.BlockSpec((1, BR, 128), lambda t, r: (t, r, 0)),
      out_shape=jax.ShapeDtypeStruct((2, NPAD, 128), F32),
  )(acc)


def _pad_rows(x, n):
  return jnp.concatenate([x, jnp.zeros((n - x.shape[0],) + x.shape[1:], x.dtype)], 0)


def _at_pad(a, b):
  m = jnp.concatenate([a, b], 0).T
  return jnp.concatenate([m, jnp.zeros((m.shape[0], 128 - m.shape[1]), F32)], 1)


def _pad_edges(src, dst):
  npd = EPAD - EDG
  si = jnp.concatenate([src, jnp.zeros((npd,), I32)])
  du = jnp.concatenate([dst, jnp.full((npd,), NU, I32)])
  return si, du


def kernel(u2i, i2u, x_user, x_item, w_user0, w_item0, au_src0, au_dst0,
           ai_src0, ai_dst0, w_user1, w_item1, au_src1, au_dst1, ai_src1,
           ai_dst1):
  xs = jnp.stack([_pad_rows(x_user, NPAD), _pad_rows(x_item, NPAD)])
  w0s = jnp.stack([w_user0, w_item0])
  at0 = jnp.stack([_at_pad(au_dst0, ai_src0), _at_pad(au_src0, ai_dst0)])
  h0, sc0 = _tc_h_scores(xs, w0s, at0)

  si0, du0 = _pad_edges(i2u[0], i2u[1])
  si1, du1 = _pad_edges(u2i[0], u2i[1])

  def run_layer(h, sc):
    return _sc_layer(
        si0, du0, si1, du1,
        sc[1, :, 0:2].reshape(-1), sc[0, :, 0:2].reshape(-1), h[1],
        sc[0, :, 2:4].reshape(-1), sc[1, :, 2:4].reshape(-1), h[0])

  acc0 = run_layer(h0, sc0)
  w1s = jnp.stack([w_user1[256:], w_item1[256:]])
  at1 = jnp.stack([_at_pad(au_dst1, ai_src1), _at_pad(au_src1, ai_dst1)])
  bar0, h1, sc1 = _tc_combine_mm(acc0, w1s, at1)

  acc1 = run_layer(h1, sc1)
  bar1 = _tc_final(acc1)

  u_bar0 = bar0[0, :NU]
  i_bar0 = bar0[1, :NI]
  u_bar1 = bar1[0, :NU]
  i_bar1 = bar1[1, :NI]
  zu = jnp.zeros((NU, D), F32)
  zi = jnp.zeros((NI, D), F32)
  u = jnp.concatenate([zu, u_bar1], axis=1)
  i = jnp.concatenate([zi, i_bar1], axis=1)
  return (u, i, u_bar0, i_bar0, u_bar1, i_bar1)
